# baseline (speedup 1.0000x reference)
import functools

import jax
import jax.numpy as jnp
from jax import lax
from jax.experimental import pallas as pl
from jax.experimental.pallas import tpu as pltpu
from jax.experimental.pallas import tpu_sc as plsc

_NUM_CLASSES = 80
_TOPK = 1000
_CONF = 0.05
_NMS_T = 0.6
_N = 20000 * _NUM_CLASSES
_NW = 32
_CHUNK = _N // _NW
_WIN = 10_000
_NWIN = _CHUNK // _WIN
_NB = 4096
_SLOT = 256
_CAND = _NW * _SLOT
_LANES = 16
_STRIDE = _NB + 1

_f32 = jnp.float32
_i32 = jnp.int32


def _lane():
    return lax.iota(_i32, _LANES)


def _key_of(x16):
    b = lax.bitcast_convert_type(x16, _i32)
    return b ^ ((b >> 31) & jnp.int32(0x7FFFFFFF))


def _hist_body(cls_hbm, hist_hbm, win_ref, stripe_ref, lhist_ref, colblk_ref,
               sp_hist):
    c = lax.axis_index("c")
    s = lax.axis_index("s")
    w = c * 16 + s
    g0 = 78 * w + jnp.minimum(w, 4)
    row0 = g0 * 8
    lane = _lane()
    zi16 = jnp.zeros((_LANES,), _i32)
    zeros16 = jnp.zeros((_LANES,), _i32)
    ones16 = jnp.ones((_LANES,), _i32)

    def zero_body(i, carry):
        for u in range(8):
            stripe_ref[pl.ds((i * 8 + u) * 16, 16)] = zeros16
        return carry

    lax.fori_loop(0, _STRIDE * 16 // (16 * 8), zero_body, 0)
    stripe_ref[pl.ds(_STRIDE * 16 - 16, 16)] = zeros16

    stripe_base = lane * _STRIDE

    def hist_rows(nrows):
        def hist_one(i, carry):
            for u in range(5):
                x = plsc.load_gather(win_ref, [zi16 + i, u * 16 + lane])
                hidx = (_key_of(x) >> 20) + jnp.int32(2048)
                plsc.addupdate_scatter(stripe_ref, [stripe_base + hidx],
                                       ones16)
            return carry

        lax.fori_loop(0, nrows, hist_one, 0)

    for wi in range(6):
        pltpu.sync_copy(
            cls_hbm.at[pl.ds(pl.multiple_of(row0 + wi * 104, 8), 104), :],
            win_ref)
        hist_rows(104)

    @pl.when(w < 4)
    def _extra_hist():
        pltpu.sync_copy(
            cls_hbm.at[pl.ds(pl.multiple_of(row0 + 624, 8), 8), :],
            win_ref.at[pl.ds(0, 8), :])
        hist_rows(8)

    def merge_body(i, carry):
        acc = stripe_ref[pl.ds(i * 16, 16)]
        for ss in range(1, 16):
            acc = acc + stripe_ref[pl.ds(ss * _STRIDE + i * 16, 16)]
        lhist_ref[pl.ds(i * 16, 16)] = acc
        return carry

    lax.fori_loop(0, _NB // 16, merge_body, 0)

    pltpu.sync_copy(lhist_ref, sp_hist.at[s])
    plsc.subcore_barrier()

    for r in range(16):
        pltpu.sync_copy(sp_hist.at[r, pl.ds(s * 256, 256)],
                        colblk_ref.at[pl.ds(r * 256, 256)])

    def merge2_body(i, carry):
        acc = colblk_ref[pl.ds(i * 16, 16)]
        for r in range(1, 16):
            acc = acc + colblk_ref[pl.ds(r * 256 + i * 16, 16)]
        lhist_ref[pl.ds(i * 16, 16)] = acc
        return carry

    lax.fori_loop(0, 256 // 16, merge2_body, 0)
    pltpu.sync_copy(lhist_ref.at[pl.ds(0, 256)],
                    hist_hbm.at[c, pl.ds(s * 256, 256)])


def _make_hist_kernel():
    mesh = plsc.VectorSubcoreMesh(core_axis_name="c", subcore_axis_name="s",
                                  num_cores=2, num_subcores=16)
    return pl.kernel(
        _hist_body,
        out_type=jax.ShapeDtypeStruct((2, _NB), _i32),
        mesh=mesh,
        compiler_params=pltpu.CompilerParams(needs_layout_passes=False),
        scratch_types=[
            pltpu.VMEM((104, 80), _f32),
            pltpu.VMEM((_STRIDE * 16,), _i32),
            pltpu.VMEM((_NB,), _i32),
            pltpu.VMEM((16 * 256,), _i32),
            pltpu.VMEM_SHARED((16, _NB), _i32),
        ],
    )


def _select_body(cls_hbm, hist_hbm, cval_hbm, cidx_hbm, win_ref, g_ref,
                 tmp_ref, cval_ref, cidx_ref):
    c = lax.axis_index("c")
    s = lax.axis_index("s")
    w = c * 16 + s
    g0 = 78 * w + jnp.minimum(w, 4)
    row0 = g0 * 8
    lane = _lane()
    zi16 = jnp.zeros((_LANES,), _i32)

    pltpu.sync_copy(hist_hbm.at[0], g_ref)
    pltpu.sync_copy(hist_hbm.at[1], tmp_ref)

    def addh(i, carry):
        g_ref[pl.ds(i * 16, 16)] = (g_ref[pl.ds(i * 16, 16)]
                                    + tmp_ref[pl.ds(i * 16, 16)])
        return carry

    lax.fori_loop(0, _NB // 16, addh, 0)

    lim = jnp.int32(_N - _TOPK)

    def scan_body(i, carry):
        pb, cnt = carry
        g16 = g_ref[pl.ds(i * 16, 16)]
        cs = plsc.cumsum(g16)
        excl = pb + cs - g16
        cond = excl <= lim
        cnt = cnt + jnp.sum(cond.astype(_i32))
        pb = pb + jnp.sum(g16)
        return pb, cnt

    _, cnt = lax.fori_loop(0, _NB // 16, scan_body,
                           (jnp.int32(0), jnp.int32(0)))
    tb = jnp.maximum(cnt - 1, 0)
    base = (tb - jnp.int32(2048)) << 20
    key_min = jnp.where(tb > 0, base - 4096, jnp.int32(-2147483647 - 1))

    sentinel = jnp.full((_LANES,), -1e30, _f32)
    zeros16 = jnp.zeros((_LANES,), _i32)

    def fill_body(i, carry):
        cval_ref[pl.ds(i * 16, 16)] = sentinel
        cidx_ref[pl.ds(i * 16, 16)] = zeros16
        return carry

    lax.fori_loop(0, (_SLOT + 16) // 16, fill_body, 0)

    off = jnp.zeros((_LANES,), _i32)

    def compact_rows(off, wrow0, nrows):
        def compact(i, off):
            for u in range(5):
                x = plsc.load_gather(win_ref, [zi16 + i, u * 16 + lane])
                m = _key_of(x) >= key_min
                offc = jnp.minimum(off, jnp.int32(_SLOT))[0]
                plsc.store_compressed(cval_ref.at[pl.ds(offc, 16)], x, mask=m)
                gidx = (wrow0 + i) * 80 + u * 16 + lane
                plsc.store_compressed(cidx_ref.at[pl.ds(offc, 16)], gidx,
                                      mask=m)
                off = off + plsc.all_reduce_population_count(m)
            return off

        return lax.fori_loop(0, nrows, compact, off)

    for wi in range(6):
        pltpu.sync_copy(
            cls_hbm.at[pl.ds(pl.multiple_of(row0 + wi * 104, 8), 104), :],
            win_ref)
        off = compact_rows(off, row0 + wi * 104, 104)

    @pl.when(w < 4)
    def _extra_compact():
        pltpu.sync_copy(
            cls_hbm.at[pl.ds(pl.multiple_of(row0 + 624, 8), 8), :],
            win_ref.at[pl.ds(0, 8), :])
        compact_rows(off, row0 + 624, 8)

    pltpu.sync_copy(cval_ref.at[pl.ds(0, _SLOT)],
                    cval_hbm.at[pl.ds(w * _SLOT, _SLOT)])
    pltpu.sync_copy(cidx_ref.at[pl.ds(0, _SLOT)],
                    cidx_hbm.at[pl.ds(w * _SLOT, _SLOT)])


def _make_select_kernel():
    mesh = plsc.VectorSubcoreMesh(core_axis_name="c", subcore_axis_name="s",
                                  num_cores=2, num_subcores=16)
    return pl.kernel(
        _select_body,
        out_type=(jax.ShapeDtypeStruct((_CAND,), _f32),
                  jax.ShapeDtypeStruct((_CAND,), _i32)),
        mesh=mesh,
        compiler_params=pltpu.CompilerParams(needs_layout_passes=False),
        scratch_types=[
            pltpu.VMEM((104, 80), _f32),
            pltpu.VMEM((_NB,), _i32),
            pltpu.VMEM((_NB,), _i32),
            pltpu.VMEM((_SLOT + 16,), _f32),
            pltpu.VMEM((_SLOT + 16,), _i32),
        ],
    )


_NP = 1024
_NSTAGE = 1008
_ROWS_PER_TILE = 63
_NWORD = _NP // 32


def _nms_body(box_hbm, anch_hbm, labf_hbm, sc_hbm, boxes_out, scores_out,
              anch_ref, labf_ref, sref, btbl_ref, bx1, bx2, bx3, bx4,
              o1, o2, o3, o4, oarea,
              mem_ref, live_ref, keep_ref, kstage_ref, bout_ref, sout_ref,
              keep_sp):
    s = lax.axis_index("s")
    lane = _lane()
    zi16 = jnp.zeros((_LANES,), _i32)
    zf16 = jnp.zeros((_LANES,), _f32)
    pw_lo = jnp.int32(1) << lane
    pw_hi = jnp.int32(1) << (lane + 16)

    pltpu.sync_copy(anch_hbm, anch_ref)
    pltpu.sync_copy(labf_hbm, labf_ref)
    pltpu.sync_copy(sc_hbm, sref.at[pl.ds(0, _NSTAGE)])
    sref[pl.ds(_NSTAGE, 16)] = zf16

    for wnd in range(4):
        pltpu.sync_copy(box_hbm.at[pl.ds(wnd * 5000, 5000), :], btbl_ref)

        def gat_body(i, carry, wnd=wnd):
            rows = anch_ref[pl.ds(i * 16, 16)]
            rel = rows - jnp.int32(wnd * 5000)
            inw = (rel >= 0) & (rel < 5000)
            relc = jnp.clip(rel, 0, 4999)
            for coord, dst in ((0, bx1), (1, bx2), (2, bx3), (3, bx4)):
                v = plsc.load_gather(btbl_ref, [relc, zi16 + coord])
                cur = dst[pl.ds(i * 16, 16)]
                dst[pl.ds(i * 16, 16)] = jnp.where(inw, v, cur)
            return carry

        lax.fori_loop(0, _NSTAGE // 16, gat_body, 0)
    for dst in (bx1, bx2, bx3, bx4):
        dst[pl.ds(_NSTAGE, 16)] = zf16

    def prep_body(i, carry):
        x1 = bx1[pl.ds(i * 16, 16)]
        y1 = bx2[pl.ds(i * 16, 16)]
        x2 = bx3[pl.ds(i * 16, 16)]
        y2 = bx4[pl.ds(i * 16, 16)]
        off = labf_ref[pl.ds(i * 16, 16)] * jnp.float32(1.0e5)
        a1 = x1 + off
        a2 = y1 + off
        a3 = x2 + off
        a4 = y2 + off
        o1[pl.ds(i * 16, 16)] = a1
        o2[pl.ds(i * 16, 16)] = a2
        o3[pl.ds(i * 16, 16)] = a3
        o4[pl.ds(i * 16, 16)] = a4
        oarea[pl.ds(i * 16, 16)] = (jnp.maximum(a3 - a1, 0.0)
                                    * jnp.maximum(a4 - a2, 0.0))
        return carry

    lax.fori_loop(0, _NSTAGE // 16, prep_body, 0)
    o1[pl.ds(_NSTAGE, 16)] = jnp.full((_LANES,), -3e30, _f32)
    o2[pl.ds(_NSTAGE, 16)] = jnp.full((_LANES,), -3e30, _f32)
    o3[pl.ds(_NSTAGE, 16)] = jnp.full((_LANES,), -4e30, _f32)
    o4[pl.ds(_NSTAGE, 16)] = jnp.full((_LANES,), -4e30, _f32)
    oarea[pl.ds(_NSTAGE, 16)] = zf16

    def zero_keep(i, carry):
        keep_ref[pl.ds(i * 16, 16)] = zf16
        return carry

    lax.fori_loop(0, _NP // 16, zero_keep, 0)

    for k in range(5):
        cf = (s * 5 + k).astype(_f32)

        def collect(i, off, k=k):
            labv = labf_ref[pl.ds(i * 16, 16)]
            m = labv == cf
            offc = jnp.minimum(off, jnp.int32(1008))[0]
            plsc.store_compressed(mem_ref.at[pl.ds(k * 1024 + offc, 16)],
                                  i * 16 + lane, mask=m)
            return off + plsc.all_reduce_population_count(m)

        offv = lax.fori_loop(0, _NSTAGE // 16, collect,
                             jnp.zeros((_LANES,), _i32))
        n0 = offv[0]
        nvreg = (n0 + 15) >> 4

        def init_live(j, carry, k=k):
            ranks = mem_ref[pl.ds(k * 1024 + j * 16, 16)]
            sc = plsc.load_gather(sref, [ranks])
            ok = (sc > jnp.float32(_CONF)) & ((j * 16 + lane) < n0)
            live_ref[pl.ds(j * 16, 16)] = ok.astype(_i32)
            return carry

        lax.fori_loop(0, nvreg, init_live, 0)

        def step(i, carry, k=k):
            alive = plsc.load_gather(live_ref, [zi16 + i])[0]

            @pl.when(alive != 0)
            def _sup():
                riv = plsc.load_gather(mem_ref, [zi16 + (k * 1024 + i)])
                rx1 = plsc.load_gather(o1, [riv])
                ry1 = plsc.load_gather(o2, [riv])
                rx2 = plsc.load_gather(o3, [riv])
                ry2 = plsc.load_gather(o4, [riv])
                rar = plsc.load_gather(oarea, [riv])

                def inner(j, cc, k=k):
                    ranks = mem_ref[pl.ds(k * 1024 + j * 16, 16)]
                    cx1 = plsc.load_gather(o1, [ranks])
                    cy1 = plsc.load_gather(o2, [ranks])
                    cx2 = plsc.load_gather(o3, [ranks])
                    cy2 = plsc.load_gather(o4, [ranks])
                    car = plsc.load_gather(oarea, [ranks])
                    xx1 = jnp.maximum(rx1, cx1)
                    yy1 = jnp.maximum(ry1, cy1)
                    xx2 = jnp.minimum(rx2, cx2)
                    yy2 = jnp.minimum(ry2, cy2)
                    inter = (jnp.maximum(xx2 - xx1, 0.0)
                             * jnp.maximum(yy2 - yy1, 0.0))
                    den = jnp.maximum(rar + car - inter, 1e-9)
                    m = ((inter > jnp.float32(_NMS_T) * den)
                         & ((j * 16 + lane) > i))
                    lv = live_ref[pl.ds(j * 16, 16)]
                    live_ref[pl.ds(j * 16, 16)] = jnp.where(m, 0, lv)
                    return cc

                lax.fori_loop(0, nvreg, inner, 0)

            return carry

        lax.fori_loop(0, n0, step, 0)

        def writeback(j, carry, k=k):
            ranks = mem_ref[pl.ds(k * 1024 + j * 16, 16)]
            lv = live_ref[pl.ds(j * 16, 16)].astype(_f32)
            plsc.store_scatter(keep_ref, [ranks], lv,
                               mask=(j * 16 + lane) < n0)
            return carry

        lax.fori_loop(0, nvreg, writeback, 0)

    pltpu.sync_copy(keep_ref, keep_sp.at[s])
    plsc.subcore_barrier()

    for r in range(16):
        pltpu.sync_copy(keep_sp.at[r, pl.ds(s * 64, 64)],
                        kstage_ref.at[pl.ds(r * 64, 64)])

    inv512 = jnp.float32(1.0 / 512.0)
    for v in range(4):
        bf = kstage_ref[pl.ds(v * 16, 16)]
        for r in range(1, 16):
            bf = bf + kstage_ref[pl.ds(r * 64 + v * 16, 16)]
        s16 = sref[pl.ds(s * 64 + v * 16, 16)]
        sout_ref[pl.ds(v * 16, 16)] = s16 * bf
        orow = v * 16 + lane
        for coord, src in ((0, bx1), (1, bx2), (2, bx3), (3, bx4)):
            raw = src[pl.ds(s * 64 + v * 16, 16)]
            bo = jnp.clip(raw * bf * inv512, 0.0, 1.0)
            plsc.store_scatter(bout_ref, [orow, zi16 + coord], bo)

    @pl.when(s < 15)
    def _store_full():
        pltpu.sync_copy(bout_ref, boxes_out.at[pl.ds(s * 64, 64), :])
        pltpu.sync_copy(sout_ref, scores_out.at[pl.ds(s * 64, 64)])

    @pl.when(s == 15)
    def _store_tail():
        pltpu.sync_copy(bout_ref.at[pl.ds(0, 40), :],
                        boxes_out.at[pl.ds(960, 40), :])
        pltpu.sync_copy(sout_ref.at[pl.ds(0, 40)],
                        scores_out.at[pl.ds(960, 40)])


def _make_nms_kernel():
    mesh = plsc.VectorSubcoreMesh(core_axis_name="c", subcore_axis_name="s",
                                  num_cores=1, num_subcores=16)
    return pl.kernel(
        _nms_body,
        out_type=(jax.ShapeDtypeStruct((_TOPK, 4), _f32),
                  jax.ShapeDtypeStruct((_TOPK,), _f32)),
        mesh=mesh,
        compiler_params=pltpu.CompilerParams(needs_layout_passes=False,
                                             use_tc_tiling_on_sc=False),
        scratch_types=[
            pltpu.VMEM((_NSTAGE,), _i32),
            pltpu.VMEM((_NSTAGE,), _f32),
            pltpu.VMEM((_NP,), _f32),
            pltpu.VMEM((5000, 4), _f32),
            pltpu.VMEM((_NP,), _f32),
            pltpu.VMEM((_NP,), _f32),
            pltpu.VMEM((_NP,), _f32),
            pltpu.VMEM((_NP,), _f32),
            pltpu.VMEM((_NP,), _f32),
            pltpu.VMEM((_NP,), _f32),
            pltpu.VMEM((_NP,), _f32),
            pltpu.VMEM((_NP,), _f32),
            pltpu.VMEM((_NP,), _f32),
            pltpu.VMEM((5 * 1024 + 16,), _i32),
            pltpu.VMEM((1024 + 16,), _i32),
            pltpu.VMEM((_NP,), _f32),
            pltpu.VMEM((_NP,), _f32),
            pltpu.VMEM((64, 4), _f32),
            pltpu.VMEM((64,), _f32),
            pltpu.VMEM_SHARED((16, _NP), _f32),
        ],
    )


_hist_kernel = _make_hist_kernel()
_select_kernel = _make_select_kernel()
_nms_kernel = _make_nms_kernel()


@jax.jit
def kernel(cls_pred, box_pred):
    cls2 = cls_pred.reshape(20000, _NUM_CLASSES)
    hist2 = _hist_kernel(cls2)
    cand_val, cand_idx = _select_kernel(cls2, hist2)

    scores = jax.nn.sigmoid(cand_val)
    topk_scores, pos = lax.top_k(scores, _TOPK)
    topk_idxs = cand_idx[pos]
    labels = topk_idxs % _NUM_CLASSES
    anchor_idxs = topk_idxs // _NUM_CLASSES

    pad_i = jnp.zeros((_NSTAGE - _TOPK,), _i32)
    anch_p = jnp.concatenate([anchor_idxs, pad_i])
    labf_p = jnp.concatenate([labels.astype(_f32),
                              jnp.full((_NSTAGE - _TOPK,), -1.0, _f32)])
    sc_p = jnp.concatenate([topk_scores, jnp.zeros((_NSTAGE - _TOPK,), _f32)])

    boxes_out, scores_out = _nms_kernel(box_pred.reshape(-1, 4), anch_p,
                                        labf_p, sc_p)
    return boxes_out, scores_out, labels


# speedup vs baseline: 1.0527x; 1.0527x over previous
import functools

import jax
import jax.numpy as jnp
from jax import lax
from jax.experimental import pallas as pl
from jax.experimental.pallas import tpu as pltpu
from jax.experimental.pallas import tpu_sc as plsc

_NUM_CLASSES = 80
_TOPK = 1000
_CONF = 0.05
_NMS_T = 0.6
_N = 20000 * _NUM_CLASSES
_NW = 32
_CHUNK = _N // _NW
_WIN = 10_000
_NWIN = _CHUNK // _WIN
_NB = 4096
_SLOT = 128
_CAND = _NW * _SLOT
_LANES = 16
_STRIDE = _NB + 1

_f32 = jnp.float32
_i32 = jnp.int32


def _lane():
    return lax.iota(_i32, _LANES)


def _key_of(x16):
    b = lax.bitcast_convert_type(x16, _i32)
    return b ^ ((b >> 31) & jnp.int32(0x7FFFFFFF))


def _hist_body(cls_hbm, hist_hbm, win_ref, stripe_ref, lhist_ref, colblk_ref,
               sp_hist):
    c = lax.axis_index("c")
    s = lax.axis_index("s")
    w = c * 16 + s
    g0 = 78 * w + jnp.minimum(w, 4)
    row0 = g0 * 8
    lane = _lane()
    zi16 = jnp.zeros((_LANES,), _i32)
    zeros16 = jnp.zeros((_LANES,), _i32)
    ones16 = jnp.ones((_LANES,), _i32)

    def zero_body(i, carry):
        for u in range(8):
            stripe_ref[pl.ds((i * 8 + u) * 16, 16)] = zeros16
        return carry

    lax.fori_loop(0, _STRIDE * 16 // (16 * 8), zero_body, 0)
    stripe_ref[pl.ds(_STRIDE * 16 - 16, 16)] = zeros16

    stripe_base = lane * _STRIDE

    def hist_rows(nrows):
        def hist_one(i, carry):
            for u in range(5):
                x = plsc.load_gather(win_ref, [zi16 + i, u * 16 + lane])
                hidx = (_key_of(x) >> 20) + jnp.int32(2048)
                plsc.addupdate_scatter(stripe_ref, [stripe_base + hidx],
                                       ones16)
            return carry

        lax.fori_loop(0, nrows, hist_one, 0)

    for wi in range(6):
        pltpu.sync_copy(
            cls_hbm.at[pl.ds(pl.multiple_of(row0 + wi * 104, 8), 104), :],
            win_ref)
        hist_rows(104)

    @pl.when(w < 4)
    def _extra_hist():
        pltpu.sync_copy(
            cls_hbm.at[pl.ds(pl.multiple_of(row0 + 624, 8), 8), :],
            win_ref.at[pl.ds(0, 8), :])
        hist_rows(8)

    def merge_body(i, carry):
        acc = stripe_ref[pl.ds(i * 16, 16)]
        for ss in range(1, 16):
            acc = acc + stripe_ref[pl.ds(ss * _STRIDE + i * 16, 16)]
        lhist_ref[pl.ds(i * 16, 16)] = acc
        return carry

    lax.fori_loop(0, _NB // 16, merge_body, 0)

    pltpu.sync_copy(lhist_ref, sp_hist.at[s])
    plsc.subcore_barrier()

    for r in range(16):
        pltpu.sync_copy(sp_hist.at[r, pl.ds(s * 256, 256)],
                        colblk_ref.at[pl.ds(r * 256, 256)])

    def merge2_body(i, carry):
        acc = colblk_ref[pl.ds(i * 16, 16)]
        for r in range(1, 16):
            acc = acc + colblk_ref[pl.ds(r * 256 + i * 16, 16)]
        lhist_ref[pl.ds(i * 16, 16)] = acc
        return carry

    lax.fori_loop(0, 256 // 16, merge2_body, 0)
    pltpu.sync_copy(lhist_ref.at[pl.ds(0, 256)],
                    hist_hbm.at[c, pl.ds(s * 256, 256)])


def _make_hist_kernel():
    mesh = plsc.VectorSubcoreMesh(core_axis_name="c", subcore_axis_name="s",
                                  num_cores=2, num_subcores=16)
    return pl.kernel(
        _hist_body,
        out_type=jax.ShapeDtypeStruct((2, _NB), _i32),
        mesh=mesh,
        compiler_params=pltpu.CompilerParams(needs_layout_passes=False),
        scratch_types=[
            pltpu.VMEM((104, 80), _f32),
            pltpu.VMEM((_STRIDE * 16,), _i32),
            pltpu.VMEM((_NB,), _i32),
            pltpu.VMEM((16 * 256,), _i32),
            pltpu.VMEM_SHARED((16, _NB), _i32),
        ],
    )


def _select_body(cls_hbm, hist_hbm, cval_hbm, cidx_hbm, win_ref, g_ref,
                 tmp_ref, cval_ref, cidx_ref):
    c = lax.axis_index("c")
    s = lax.axis_index("s")
    w = c * 16 + s
    g0 = 78 * w + jnp.minimum(w, 4)
    row0 = g0 * 8
    lane = _lane()
    zi16 = jnp.zeros((_LANES,), _i32)

    pltpu.sync_copy(hist_hbm.at[0], g_ref)
    pltpu.sync_copy(hist_hbm.at[1], tmp_ref)

    def addh(i, carry):
        g_ref[pl.ds(i * 16, 16)] = (g_ref[pl.ds(i * 16, 16)]
                                    + tmp_ref[pl.ds(i * 16, 16)])
        return carry

    lax.fori_loop(0, _NB // 16, addh, 0)

    lim = jnp.int32(_N - _TOPK)

    def scan_body(i, carry):
        pb, cnt = carry
        g16 = g_ref[pl.ds(i * 16, 16)]
        cs = plsc.cumsum(g16)
        excl = pb + cs - g16
        cond = excl <= lim
        cnt = cnt + jnp.sum(cond.astype(_i32))
        pb = pb + jnp.sum(g16)
        return pb, cnt

    _, cnt = lax.fori_loop(0, _NB // 16, scan_body,
                           (jnp.int32(0), jnp.int32(0)))
    tb = jnp.maximum(cnt - 1, 0)
    base = (tb - jnp.int32(2048)) << 20
    key_min = jnp.where(tb > 0, base - 4096, jnp.int32(-2147483647 - 1))

    sentinel = jnp.full((_LANES,), -1e30, _f32)
    zeros16 = jnp.zeros((_LANES,), _i32)

    def fill_body(i, carry):
        cval_ref[pl.ds(i * 16, 16)] = sentinel
        cidx_ref[pl.ds(i * 16, 16)] = zeros16
        return carry

    lax.fori_loop(0, (_SLOT + 16) // 16, fill_body, 0)

    off = jnp.zeros((_LANES,), _i32)

    def compact_rows(off, wrow0, nrows):
        def compact(i, off):
            for u in range(5):
                x = plsc.load_gather(win_ref, [zi16 + i, u * 16 + lane])
                m = _key_of(x) >= key_min
                offc = jnp.minimum(off, jnp.int32(_SLOT))[0]
                plsc.store_compressed(cval_ref.at[pl.ds(offc, 16)], x, mask=m)
                gidx = (wrow0 + i) * 80 + u * 16 + lane
                plsc.store_compressed(cidx_ref.at[pl.ds(offc, 16)], gidx,
                                      mask=m)
                off = off + plsc.all_reduce_population_count(m)
            return off

        return lax.fori_loop(0, nrows, compact, off)

    for wi in range(6):
        pltpu.sync_copy(
            cls_hbm.at[pl.ds(pl.multiple_of(row0 + wi * 104, 8), 104), :],
            win_ref)
        off = compact_rows(off, row0 + wi * 104, 104)

    @pl.when(w < 4)
    def _extra_compact():
        pltpu.sync_copy(
            cls_hbm.at[pl.ds(pl.multiple_of(row0 + 624, 8), 8), :],
            win_ref.at[pl.ds(0, 8), :])
        compact_rows(off, row0 + 624, 8)

    pltpu.sync_copy(cval_ref.at[pl.ds(0, _SLOT)],
                    cval_hbm.at[pl.ds(w * _SLOT, _SLOT)])
    pltpu.sync_copy(cidx_ref.at[pl.ds(0, _SLOT)],
                    cidx_hbm.at[pl.ds(w * _SLOT, _SLOT)])


def _make_select_kernel():
    mesh = plsc.VectorSubcoreMesh(core_axis_name="c", subcore_axis_name="s",
                                  num_cores=2, num_subcores=16)
    return pl.kernel(
        _select_body,
        out_type=(jax.ShapeDtypeStruct((_CAND,), _f32),
                  jax.ShapeDtypeStruct((_CAND,), _i32)),
        mesh=mesh,
        compiler_params=pltpu.CompilerParams(needs_layout_passes=False),
        scratch_types=[
            pltpu.VMEM((104, 80), _f32),
            pltpu.VMEM((_NB,), _i32),
            pltpu.VMEM((_NB,), _i32),
            pltpu.VMEM((_SLOT + 16,), _f32),
            pltpu.VMEM((_SLOT + 16,), _i32),
        ],
    )


_NP = 1024
_NSTAGE = 1008
_ROWS_PER_TILE = 63
_NWORD = _NP // 32


def _nms_body(box_hbm, anch_hbm, labf_hbm, sc_hbm, boxes_out, scores_out,
              anch_ref, labf_ref, sref, btbl_ref, bx1, bx2, bx3, bx4,
              o1, o2, o3, o4, oarea,
              mem_ref, live_ref, keep_ref, kstage_ref, bout_ref, sout_ref,
              keep_sp):
    s = lax.axis_index("s")
    lane = _lane()
    zi16 = jnp.zeros((_LANES,), _i32)
    zf16 = jnp.zeros((_LANES,), _f32)
    pw_lo = jnp.int32(1) << lane
    pw_hi = jnp.int32(1) << (lane + 16)

    pltpu.sync_copy(anch_hbm, anch_ref)
    pltpu.sync_copy(labf_hbm, labf_ref)
    pltpu.sync_copy(sc_hbm, sref.at[pl.ds(0, _NSTAGE)])
    sref[pl.ds(_NSTAGE, 16)] = zf16

    for wnd, (wst, wlen) in enumerate(((0, 160), (160, 160), (320, 160),
                                       (464, 161))):
        pltpu.sync_copy(box_hbm.at[pl.ds(wst, wlen), :],
                        btbl_ref.at[pl.ds(0, wlen), :])

        def gat_body(i, carry, wst=wst, wlen=wlen):
            rows = anch_ref[pl.ds(i * 16, 16)]
            brow = rows >> 5
            bcol = (rows & 31) << 2
            rel = brow - jnp.int32(wst)
            inw = (rel >= 0) & (rel < wlen)
            relc = jnp.clip(rel, 0, wlen - 1)
            for coord, dst in ((0, bx1), (1, bx2), (2, bx3), (3, bx4)):
                v = plsc.load_gather(btbl_ref, [relc, bcol + coord])
                cur = dst[pl.ds(i * 16, 16)]
                dst[pl.ds(i * 16, 16)] = jnp.where(inw, v, cur)
            return carry

        lax.fori_loop(0, _NSTAGE // 16, gat_body, 0)
    for dst in (bx1, bx2, bx3, bx4):
        dst[pl.ds(_NSTAGE, 16)] = zf16

    def prep_body(i, carry):
        x1 = bx1[pl.ds(i * 16, 16)]
        y1 = bx2[pl.ds(i * 16, 16)]
        x2 = bx3[pl.ds(i * 16, 16)]
        y2 = bx4[pl.ds(i * 16, 16)]
        off = labf_ref[pl.ds(i * 16, 16)] * jnp.float32(1.0e5)
        a1 = x1 + off
        a2 = y1 + off
        a3 = x2 + off
        a4 = y2 + off
        o1[pl.ds(i * 16, 16)] = a1
        o2[pl.ds(i * 16, 16)] = a2
        o3[pl.ds(i * 16, 16)] = a3
        o4[pl.ds(i * 16, 16)] = a4
        oarea[pl.ds(i * 16, 16)] = (jnp.maximum(a3 - a1, 0.0)
                                    * jnp.maximum(a4 - a2, 0.0))
        return carry

    lax.fori_loop(0, _NSTAGE // 16, prep_body, 0)
    o1[pl.ds(_NSTAGE, 16)] = jnp.full((_LANES,), -3e30, _f32)
    o2[pl.ds(_NSTAGE, 16)] = jnp.full((_LANES,), -3e30, _f32)
    o3[pl.ds(_NSTAGE, 16)] = jnp.full((_LANES,), -4e30, _f32)
    o4[pl.ds(_NSTAGE, 16)] = jnp.full((_LANES,), -4e30, _f32)
    oarea[pl.ds(_NSTAGE, 16)] = zf16

    def zero_keep(i, carry):
        keep_ref[pl.ds(i * 16, 16)] = zf16
        return carry

    lax.fori_loop(0, _NP // 16, zero_keep, 0)

    for k in range(5):
        cf = (s * 5 + k).astype(_f32)

        def collect(i, off, k=k):
            labv = labf_ref[pl.ds(i * 16, 16)]
            m = labv == cf
            offc = jnp.minimum(off, jnp.int32(1008))[0]
            plsc.store_compressed(mem_ref.at[pl.ds(k * 1024 + offc, 16)],
                                  i * 16 + lane, mask=m)
            return off + plsc.all_reduce_population_count(m)

        offv = lax.fori_loop(0, _NSTAGE // 16, collect,
                             jnp.zeros((_LANES,), _i32))
        n0 = offv[0]
        nvreg = (n0 + 15) >> 4

        def init_live(j, carry, k=k):
            ranks = mem_ref[pl.ds(k * 1024 + j * 16, 16)]
            sc = plsc.load_gather(sref, [ranks])
            ok = (sc > jnp.float32(_CONF)) & ((j * 16 + lane) < n0)
            live_ref[pl.ds(j * 16, 16)] = ok.astype(_i32)
            return carry

        lax.fori_loop(0, nvreg, init_live, 0)

        def step(i, carry, k=k):
            alive = plsc.load_gather(live_ref, [zi16 + i])[0]

            @pl.when(alive != 0)
            def _sup():
                riv = plsc.load_gather(mem_ref, [zi16 + (k * 1024 + i)])
                rx1 = plsc.load_gather(o1, [riv])
                ry1 = plsc.load_gather(o2, [riv])
                rx2 = plsc.load_gather(o3, [riv])
                ry2 = plsc.load_gather(o4, [riv])
                rar = plsc.load_gather(oarea, [riv])

                def inner(j, cc, k=k):
                    ranks = mem_ref[pl.ds(k * 1024 + j * 16, 16)]
                    cx1 = plsc.load_gather(o1, [ranks])
                    cy1 = plsc.load_gather(o2, [ranks])
                    cx2 = plsc.load_gather(o3, [ranks])
                    cy2 = plsc.load_gather(o4, [ranks])
                    car = plsc.load_gather(oarea, [ranks])
                    xx1 = jnp.maximum(rx1, cx1)
                    yy1 = jnp.maximum(ry1, cy1)
                    xx2 = jnp.minimum(rx2, cx2)
                    yy2 = jnp.minimum(ry2, cy2)
                    inter = (jnp.maximum(xx2 - xx1, 0.0)
                             * jnp.maximum(yy2 - yy1, 0.0))
                    den = jnp.maximum(rar + car - inter, 1e-9)
                    m = ((inter > jnp.float32(_NMS_T) * den)
                         & ((j * 16 + lane) > i))
                    lv = live_ref[pl.ds(j * 16, 16)]
                    live_ref[pl.ds(j * 16, 16)] = jnp.where(m, 0, lv)
                    return cc

                lax.fori_loop(0, nvreg, inner, 0)

            return carry

        lax.fori_loop(0, n0, step, 0)

        def writeback(j, carry, k=k):
            ranks = mem_ref[pl.ds(k * 1024 + j * 16, 16)]
            lv = live_ref[pl.ds(j * 16, 16)].astype(_f32)
            plsc.store_scatter(keep_ref, [ranks], lv,
                               mask=(j * 16 + lane) < n0)
            return carry

        lax.fori_loop(0, nvreg, writeback, 0)

    pltpu.sync_copy(keep_ref, keep_sp.at[s])
    plsc.subcore_barrier()

    for r in range(16):
        pltpu.sync_copy(keep_sp.at[r, pl.ds(s * 64, 64)],
                        kstage_ref.at[pl.ds(r * 64, 64)])

    inv512 = jnp.float32(1.0 / 512.0)
    for v in range(4):
        bf = kstage_ref[pl.ds(v * 16, 16)]
        for r in range(1, 16):
            bf = bf + kstage_ref[pl.ds(r * 64 + v * 16, 16)]
        s16 = sref[pl.ds(s * 64 + v * 16, 16)]
        sout_ref[pl.ds(v * 16, 16)] = s16 * bf
        orow = v * 16 + lane
        for coord, src in ((0, bx1), (1, bx2), (2, bx3), (3, bx4)):
            raw = src[pl.ds(s * 64 + v * 16, 16)]
            bo = jnp.clip(raw * bf * inv512, 0.0, 1.0)
            plsc.store_scatter(bout_ref, [orow, zi16 + coord], bo)

    @pl.when(s < 15)
    def _store_full():
        pltpu.sync_copy(bout_ref, boxes_out.at[pl.ds(s * 64, 64), :])
        pltpu.sync_copy(sout_ref, scores_out.at[pl.ds(s * 64, 64)])

    @pl.when(s == 15)
    def _store_tail():
        pltpu.sync_copy(bout_ref.at[pl.ds(0, 40), :],
                        boxes_out.at[pl.ds(960, 40), :])
        pltpu.sync_copy(sout_ref.at[pl.ds(0, 40)],
                        scores_out.at[pl.ds(960, 40)])


def _make_nms_kernel():
    mesh = plsc.VectorSubcoreMesh(core_axis_name="c", subcore_axis_name="s",
                                  num_cores=1, num_subcores=16)
    return pl.kernel(
        _nms_body,
        out_type=(jax.ShapeDtypeStruct((_TOPK, 4), _f32),
                  jax.ShapeDtypeStruct((_TOPK,), _f32)),
        mesh=mesh,
        compiler_params=pltpu.CompilerParams(needs_layout_passes=False,
                                             use_tc_tiling_on_sc=False),
        scratch_types=[
            pltpu.VMEM((_NSTAGE,), _i32),
            pltpu.VMEM((_NSTAGE,), _f32),
            pltpu.VMEM((_NP,), _f32),
            pltpu.VMEM((161, 128), _f32),
            pltpu.VMEM((_NP,), _f32),
            pltpu.VMEM((_NP,), _f32),
            pltpu.VMEM((_NP,), _f32),
            pltpu.VMEM((_NP,), _f32),
            pltpu.VMEM((_NP,), _f32),
            pltpu.VMEM((_NP,), _f32),
            pltpu.VMEM((_NP,), _f32),
            pltpu.VMEM((_NP,), _f32),
            pltpu.VMEM((_NP,), _f32),
            pltpu.VMEM((5 * 1024 + 16,), _i32),
            pltpu.VMEM((1024 + 16,), _i32),
            pltpu.VMEM((_NP,), _f32),
            pltpu.VMEM((_NP,), _f32),
            pltpu.VMEM((64, 4), _f32),
            pltpu.VMEM((64,), _f32),
            pltpu.VMEM_SHARED((16, _NP), _f32),
        ],
    )


_hist_kernel = _make_hist_kernel()
_select_kernel = _make_select_kernel()
_nms_kernel = _make_nms_kernel()


@jax.jit
def kernel(cls_pred, box_pred):
    cls2 = cls_pred.reshape(20000, _NUM_CLASSES)
    hist2 = _hist_kernel(cls2)
    cand_val, cand_idx = _select_kernel(cls2, hist2)

    scores = jax.nn.sigmoid(cand_val)
    topk_scores, pos = lax.top_k(scores, _TOPK)
    topk_idxs = cand_idx[pos]
    labels = topk_idxs % _NUM_CLASSES
    anchor_idxs = topk_idxs // _NUM_CLASSES

    pad_i = jnp.zeros((_NSTAGE - _TOPK,), _i32)
    anch_p = jnp.concatenate([anchor_idxs, pad_i])
    labf_p = jnp.concatenate([labels.astype(_f32),
                              jnp.full((_NSTAGE - _TOPK,), -1.0, _f32)])
    sc_p = jnp.concatenate([topk_scores, jnp.zeros((_NSTAGE - _TOPK,), _f32)])

    boxes_out, scores_out = _nms_kernel(box_pred.reshape(625, 128), anch_p,
                                        labf_p, sc_p)
    return boxes_out, scores_out, labels
